# baseline (speedup 1.0000x reference)
import functools

import jax
import jax.numpy as jnp
from jax import lax
from jax.experimental import pallas as pl
from jax.experimental.pallas import tpu as pltpu
from jax.experimental.pallas import tpu_sc as plsc

_N = 50000
_E = 800000
_H = 64
_R = 50
_CU = 5.0

_NC = 2
_NS = 16
_B = 256
_EPAD = 819200
_PAD = _EPAD - _E
_EPT = _EPAD // _NS
_NP = 50048
_RPT = _NP // _NS

_BE = 4096
_GE = _EPAD // _BE
_BN = 2000
_GN = _N // _BN


def _filter1_body(d_ref, means_ref, betas_ref, w_ref, b_ref, o_ref):
    d = d_ref[0, 0, :]
    cut = 0.5 * (jnp.cos(d * (jnp.pi / _CU)) + 1.0) * (d < _CU).astype(jnp.float32)
    ea = cut[:, None] * jnp.exp(
        -betas_ref[0, :][None, :] * (jnp.exp(-d)[:, None] - means_ref[0, :][None, :]) ** 2)
    h = jnp.dot(ea, w_ref[...], preferred_element_type=jnp.float32) + b_ref[0, :][None, :]
    wf = h * cut[:, None]
    o_ref[0] = wf[:, :32]
    o_ref[1] = wf[:, 32:]


def _filter2_body(d_ref, means_ref, betas_ref, w1_ref, b1_ref, w2_ref, b2_ref, o_ref):
    d = d_ref[0, 0, :]
    cut = 0.5 * (jnp.cos(d * (jnp.pi / _CU)) + 1.0) * (d < _CU).astype(jnp.float32)
    ea = cut[:, None] * jnp.exp(
        -betas_ref[0, :][None, :] * (jnp.exp(-d)[:, None] - means_ref[0, :][None, :]) ** 2)
    h = jnp.dot(ea, w1_ref[...], preferred_element_type=jnp.float32) + b1_ref[0, :][None, :]
    h = jax.nn.silu(h)
    h = jnp.dot(h, w2_ref[...], preferred_element_type=jnp.float32) + b2_ref[0, :][None, :]
    wf = h * cut[:, None]
    o_ref[0] = wf[:, :32]
    o_ref[1] = wf[:, 32:]


_w_spec = lambda shape: pl.BlockSpec(shape, lambda g: (0,) * len(shape))
_d_spec = pl.BlockSpec((1, 1, _BE), lambda g: (g, 0, 0))
_wf_spec = pl.BlockSpec((2, _BE, 32), lambda g: (0, g, 0))
_wf_shape = jax.ShapeDtypeStruct((2, _EPAD, 32), jnp.float32)

_filter1 = pl.pallas_call(
    _filter1_body, grid=(_GE,),
    in_specs=[_d_spec, _w_spec((1, _R)), _w_spec((1, _R)),
              _w_spec((_R, _H)), _w_spec((1, _H))],
    out_specs=_wf_spec, out_shape=_wf_shape)

_filter2 = pl.pallas_call(
    _filter2_body, grid=(_GE,),
    in_specs=[_d_spec, _w_spec((1, _R)), _w_spec((1, _R)),
              _w_spec((_R, _H)), _w_spec((1, _H)),
              _w_spec((_H, _H)), _w_spec((1, _H))],
    out_specs=_wf_spec, out_shape=_wf_shape)


def _embed_body(z_ref, emb_ref, emb2_ref, x0_ref, xn_ref):
    zb = z_ref[0, 0, :]
    oh = (zb[:, None] == lax.broadcasted_iota(jnp.int32, (_BN, 100), 1)).astype(jnp.float32)
    x0_ref[...] = jnp.dot(oh, emb_ref[...], preferred_element_type=jnp.float32)
    xn = jnp.dot(oh, emb2_ref[...], preferred_element_type=jnp.float32)
    xn_ref[0] = xn[:, :32]
    xn_ref[1] = xn[:, 32:]


_embed = pl.pallas_call(
    _embed_body, grid=(_GN,),
    in_specs=[pl.BlockSpec((1, 1, _BN), lambda g: (g, 0, 0)),
              _w_spec((100, _H)), _w_spec((100, _H))],
    out_specs=[pl.BlockSpec((_BN, _H), lambda g: (g, 0)),
               pl.BlockSpec((2, _BN, 32), lambda g: (0, g, 0))],
    out_shape=[jax.ShapeDtypeStruct((_N, _H), jnp.float32),
               jax.ShapeDtypeStruct((2, _N, 32), jnp.float32)])


def _comb_body(x0_ref, agg_ref, cw_ref, cb_ref, l1_ref, x_ref, x1_ref):
    cat = jnp.concatenate([x0_ref[...], agg_ref[0], agg_ref[1]], axis=1)
    xb = jnp.dot(cat, cw_ref[...], preferred_element_type=jnp.float32) + cb_ref[0, :][None, :]
    x_ref[...] = xb
    x1 = jnp.dot(xb, l1_ref[...], preferred_element_type=jnp.float32)
    x1_ref[0] = x1[:, :32]
    x1_ref[1] = x1[:, 32:]


_comb = pl.pallas_call(
    _comb_body, grid=(_GN,),
    in_specs=[pl.BlockSpec((_BN, _H), lambda g: (g, 0)),
              pl.BlockSpec((2, _BN, 32), lambda g: (0, g, 0)),
              _w_spec((2 * _H, _H)), _w_spec((1, _H)), _w_spec((_H, _H))],
    out_specs=[pl.BlockSpec((_BN, _H), lambda g: (g, 0)),
               pl.BlockSpec((2, _BN, 32), lambda g: (0, g, 0))],
    out_shape=[jax.ShapeDtypeStruct((_N, _H), jnp.float32),
               jax.ShapeDtypeStruct((2, _N, 32), jnp.float32)])


def _layer_body(x_ref, s_ref, hist_ref, l2_ref, l2b_ref, lw_ref, lwb_ref, *rest,
                has_next):
    if has_next:
        l1n_ref, x_out, x1_out = rest
    else:
        (x_out,) = rest
    cnt = jnp.clip(hist_ref[0, :, 0] + hist_ref[1, :, 0], 1.0, None)
    sm = jnp.concatenate([s_ref[0], s_ref[1]], axis=1) / cnt[:, None]
    v = jnp.dot(sm, l2_ref[...], preferred_element_type=jnp.float32) + l2b_ref[0, :][None, :]
    v = jax.nn.silu(v)
    v = jnp.dot(v, lw_ref[...], preferred_element_type=jnp.float32) + lwb_ref[0, :][None, :]
    xn = x_ref[...] + v
    x_out[...] = xn
    if has_next:
        x1 = jnp.dot(xn, l1n_ref[...], preferred_element_type=jnp.float32)
        x1_out[0] = x1[:, :32]
        x1_out[1] = x1[:, 32:]


def _make_layer(has_next):
    in_specs = [pl.BlockSpec((_BN, _H), lambda g: (g, 0)),
                pl.BlockSpec((2, _BN, 32), lambda g: (0, g, 0)),
                pl.BlockSpec((2, _BN, 8), lambda g: (0, g, 0)),
                _w_spec((_H, _H)), _w_spec((1, _H)),
                _w_spec((_H, _H)), _w_spec((1, _H))]
    out_specs = [pl.BlockSpec((_BN, _H), lambda g: (g, 0))]
    out_shape = [jax.ShapeDtypeStruct((_N, _H), jnp.float32)]
    if has_next:
        in_specs.append(_w_spec((_H, _H)))
        out_specs.append(pl.BlockSpec((2, _BN, 32), lambda g: (0, g, 0)))
        out_shape.append(jax.ShapeDtypeStruct((2, _N, 32), jnp.float32))
    return pl.pallas_call(
        functools.partial(_layer_body, has_next=has_next), grid=(_GN,),
        in_specs=in_specs, out_specs=out_specs, out_shape=out_shape)


_layer_next = _make_layer(True)
_layer_last = _make_layer(False)


_SC_PARAMS = pltpu.CompilerParams(needs_layout_passes=False,
                                  use_tc_tiling_on_sc=False)


def _make_sc_scatter():
    mesh = plsc.VectorSubcoreMesh(core_axis_name="c", subcore_axis_name="s",
                                  num_cores=_NC)
    out_type = jax.ShapeDtypeStruct((_NC * _NP, 32), jnp.float32)
    scratch = [
        pltpu.VMEM((8, 128), jnp.int32),
        pltpu.VMEM((8, 128), jnp.int32),
        pltpu.VMEM((2, 128, 32), jnp.float32),
        pltpu.VMEM((2, 128, 32), jnp.float32),
        pltpu.VMEM_SHARED((_NP, 32), jnp.float32),
        pltpu.SemaphoreType.DMA, pltpu.SemaphoreType.DMA,
        pltpu.SemaphoreType.DMA, pltpu.SemaphoreType.DMA,
        pltpu.SemaphoreType.DMA, pltpu.SemaphoreType.DMA,
    ]

    def body(src_hbm, dst_hbm, wf_hbm, table_hbm, zeros_hbm, out_hbm,
             srcv, dstv, gath, wfv, acc, sg0, sg1, sw0, sw1, ss0, ss1):
        c = lax.axis_index("c")
        t = lax.axis_index("s")
        sgs, sws, sss = [sg0, sg1], [sw0, sw1], [ss0, ss1]

        pltpu.sync_copy(zeros_hbm, acc.at[pl.ds(t * _RPT, _RPT)])
        plsc.subcore_barrier()

        idx_row0 = t * (_EPT // 128)

        def grp(i, carry):
            rb = idx_row0 + i * 8
            pltpu.sync_copy(src_hbm.at[pl.ds(c * (_EPAD // 128) + rb, 8)], srcv)
            pltpu.sync_copy(dst_hbm.at[pl.ds(rb, 8)], dstv)
            gbase = t * _EPT + i * 1024

            def issue(sb):
                s = sb % 2
                ebase = gbase + sb * 128
                hw = pltpu.async_copy(wf_hbm.at[c, pl.ds(ebase, 128)],
                                      wfv.at[s], sws[s])
                hg = pltpu.async_copy(table_hbm.at[srcv.at[sb]],
                                      gath.at[s], sgs[s])
                return hw, hg

            hws, hgs = [None] * 8, [None] * 8
            hss = [None] * 8
            hws[0], hgs[0] = issue(0)
            for sb in range(8):
                s = sb % 2
                if sb < 7:
                    if sb >= 1:
                        hss[sb - 1].wait()
                    hws[sb + 1], hgs[sb + 1] = issue(sb + 1)
                hws[sb].wait()
                hgs[sb].wait()
                gslot, wslot = gath.at[s], wfv.at[s]

                @plsc.parallel_loop(0, 128, unroll=8)
                def _(r):
                    gslot[r, pl.ds(0, 16)] = (gslot[r, pl.ds(0, 16)]
                                              * wslot[r, pl.ds(0, 16)])
                    gslot[r, pl.ds(16, 16)] = (gslot[r, pl.ds(16, 16)]
                                               * wslot[r, pl.ds(16, 16)])

                hss[sb] = pltpu.async_copy(gath.at[s], acc.at[dstv.at[sb]],
                                           sss[s], add=True)
            hss[6].wait()
            hss[7].wait()
            return carry

        lax.fori_loop(0, _EPT // 1024, grp, 0)
        plsc.subcore_barrier()
        pltpu.sync_copy(acc.at[pl.ds(t * _RPT, _RPT)],
                        out_hbm.at[pl.ds(c * _NP + t * _RPT, _RPT)])

    return pl.kernel(body, out_type=out_type, mesh=mesh, scratch_types=scratch,
                     compiler_params=_SC_PARAMS)


def _make_sc_degree():
    mesh = plsc.VectorSubcoreMesh(core_axis_name="c", subcore_axis_name="s",
                                  num_cores=_NC)
    out_type = jax.ShapeDtypeStruct((_NC * _NP, 8), jnp.float32)
    scratch = [
        pltpu.VMEM((8, 128), jnp.int32),
        pltpu.VMEM((128, 8), jnp.float32),
        pltpu.VMEM_SHARED((_NP, 8), jnp.float32),
        pltpu.SemaphoreType.DMA, pltpu.SemaphoreType.DMA,
        pltpu.SemaphoreType.DMA, pltpu.SemaphoreType.DMA,
    ]
    rows_per_worker = (_EPAD // 128) // (_NC * _NS)

    def body(dst_hbm, zeros8_hbm, ones_hbm, out_hbm, dstv, ones, hist,
             s0, s1, s2, s3):
        c = lax.axis_index("c")
        t = lax.axis_index("s")
        sems = [s0, s1, s2, s3]
        pltpu.sync_copy(zeros8_hbm, hist.at[pl.ds(t * _RPT, _RPT)])
        pltpu.sync_copy(ones_hbm, ones)
        plsc.subcore_barrier()

        row0 = (c * _NS + t) * rows_per_worker

        def grp(i, carry):
            pltpu.sync_copy(dst_hbm.at[pl.ds(row0 + i * 8, 8)], dstv)
            hs = [pltpu.async_copy(ones, hist.at[dstv.at[j]], sems[j % 4],
                                   add=True)
                  for j in range(8)]
            for h in hs:
                h.wait()
            return carry

        lax.fori_loop(0, rows_per_worker // 8, grp, 0)
        plsc.subcore_barrier()
        pltpu.sync_copy(hist.at[pl.ds(t * _RPT, _RPT)],
                        out_hbm.at[pl.ds(c * _NP + t * _RPT, _RPT)])

    return pl.kernel(body, out_type=out_type, mesh=mesh, scratch_types=scratch,
                     compiler_params=_SC_PARAMS)


_sc_scatter = _make_sc_scatter()
_sc_degree = _make_sc_degree()


def kernel(z, edge_index, edge_weight, emb, emb2, means, betas, dp_W, dp_b,
           comb_W, comb_b, mlp1_W, mlp1_b, mlp2_W, mlp2_b,
           lin1_W, lin2_W, lin2_b, lin_W, lin_b):
    f32 = jnp.float32
    src = edge_index[0].astype(jnp.int32)
    dst = edge_index[1].astype(jnp.int32)
    pad_idx = (jnp.arange(_PAD, dtype=jnp.int32) * 97) % _N
    src_p = jnp.concatenate([src, pad_idx])
    dst_p = jnp.concatenate([dst, jnp.full((_PAD,), _N, jnp.int32)])
    src2 = jnp.concatenate([src_p, src_p + _N]).reshape(2 * _EPAD // 128, 128)
    dst2 = dst_p.reshape(_EPAD // 128, 128)
    d_pad = jnp.concatenate(
        [edge_weight.astype(f32), jnp.full((_PAD,), 7.0, f32)]).reshape(_GE, 1, _BE)
    zeros2d = jnp.zeros((_RPT, 32), f32)
    zeros8 = jnp.zeros((_RPT, 8), f32)
    ones8 = jnp.ones((128, 8), f32)
    means2 = means.reshape(1, _R)
    betas2 = betas.reshape(1, _R)

    x0, xn = _embed(z.astype(jnp.int32).reshape(_GN, 1, _BN), emb, emb2)

    wn = _filter1(d_pad, means2, betas2, dp_W, dp_b.reshape(1, _H))
    agg_flat = _sc_scatter(src2, dst2, wn, xn.reshape(2 * _N, 32), zeros2d)
    agg = agg_flat.reshape(2, _NP, 32)[:, :_N, :]
    hist = _sc_degree(dst2, zeros8, ones8)
    hist3 = hist.reshape(2, _NP, 8)[:, :_N, :]

    x, x1 = _comb(x0, agg, comb_W, comb_b.reshape(1, _H), lin1_W[0])

    for l in range(4):
        wf = _filter2(d_pad, means2, betas2, mlp1_W[l], mlp1_b[l].reshape(1, _H),
                      mlp2_W[l], mlp2_b[l].reshape(1, _H))
        s_flat = _sc_scatter(src2, dst2, wf, x1.reshape(2 * _N, 32), zeros2d)
        s = s_flat.reshape(2, _NP, 32)[:, :_N, :]
        common = (x, s, hist3, lin2_W[l], lin2_b[l].reshape(1, _H),
                  lin_W[l], lin_b[l].reshape(1, _H))
        if l < 3:
            x, x1 = _layer_next(*common, lin1_W[l + 1])
        else:
            x, = _layer_last(*common)
    return x


# speedup vs baseline: 1.4248x; 1.4248x over previous
import functools

import jax
import jax.numpy as jnp
from jax import lax
from jax.experimental import pallas as pl
from jax.experimental.pallas import tpu as pltpu
from jax.experimental.pallas import tpu_sc as plsc

_N = 50000
_E = 800000
_H = 64
_R = 50
_CU = 5.0

_NC = 2
_NS = 16
_B = 256
_EPAD = 819200
_PAD = _EPAD - _E
_EPT = _EPAD // _NS
_NP = 50048
_RPT = _NP // _NS

_BE = 1024
_GE = _EPAD // _BE
_BN = 2000
_GN = _N // _BN


def _filter1_body(d_ref, means_ref, betas_ref, w_ref, b_ref, o_ref):
    d = d_ref[0, 0, :]
    cut = 0.5 * (jnp.cos(d * (jnp.pi / _CU)) + 1.0) * (d < _CU).astype(jnp.float32)
    ea = cut[:, None] * jnp.exp(
        -betas_ref[0, :][None, :] * (jnp.exp(-d)[:, None] - means_ref[0, :][None, :]) ** 2)
    h = jnp.dot(ea, w_ref[...], preferred_element_type=jnp.float32) + b_ref[0, :][None, :]
    wf = h * cut[:, None]
    o_ref[0] = jnp.concatenate(
        [wf[256 * k:256 * (k + 1), :32] for k in range(4)], axis=1)
    o_ref[1] = jnp.concatenate(
        [wf[256 * k:256 * (k + 1), 32:] for k in range(4)], axis=1)


def _filter2_body(d_ref, means_ref, betas_ref, w1_ref, b1_ref, w2_ref, b2_ref, o_ref):
    d = d_ref[0, 0, :]
    cut = 0.5 * (jnp.cos(d * (jnp.pi / _CU)) + 1.0) * (d < _CU).astype(jnp.float32)
    ea = cut[:, None] * jnp.exp(
        -betas_ref[0, :][None, :] * (jnp.exp(-d)[:, None] - means_ref[0, :][None, :]) ** 2)
    h = jnp.dot(ea, w1_ref[...], preferred_element_type=jnp.float32) + b1_ref[0, :][None, :]
    h = jax.nn.silu(h)
    h = jnp.dot(h, w2_ref[...], preferred_element_type=jnp.float32) + b2_ref[0, :][None, :]
    wf = h * cut[:, None]
    o_ref[0] = jnp.concatenate(
        [wf[256 * k:256 * (k + 1), :32] for k in range(4)], axis=1)
    o_ref[1] = jnp.concatenate(
        [wf[256 * k:256 * (k + 1), 32:] for k in range(4)], axis=1)


_w_spec = lambda shape: pl.BlockSpec(shape, lambda g: (0,) * len(shape))
_d_spec = pl.BlockSpec((1, 1, _BE), lambda g: (g, 0, 0))
_wf_spec = pl.BlockSpec((2, _BE // 4, 128), lambda g: (0, g, 0))
_wf_shape = jax.ShapeDtypeStruct((2, _EPAD // 4, 128), jnp.float32)

_filter1 = pl.pallas_call(
    _filter1_body, grid=(_GE,),
    in_specs=[_d_spec, _w_spec((1, _R)), _w_spec((1, _R)),
              _w_spec((_R, _H)), _w_spec((1, _H))],
    out_specs=_wf_spec, out_shape=_wf_shape)

_filter2 = pl.pallas_call(
    _filter2_body, grid=(_GE,),
    in_specs=[_d_spec, _w_spec((1, _R)), _w_spec((1, _R)),
              _w_spec((_R, _H)), _w_spec((1, _H)),
              _w_spec((_H, _H)), _w_spec((1, _H))],
    out_specs=_wf_spec, out_shape=_wf_shape)


def _embed_body(z_ref, emb_ref, emb2_ref, x0_ref, xn_ref):
    zb = z_ref[0, 0, :]
    oh = (zb[:, None] == lax.broadcasted_iota(jnp.int32, (_BN, 100), 1)).astype(jnp.float32)
    x0_ref[...] = jnp.dot(oh, emb_ref[...], preferred_element_type=jnp.float32)
    xn = jnp.dot(oh, emb2_ref[...], preferred_element_type=jnp.float32)
    xn_ref[0] = xn[:, :32]
    xn_ref[1] = xn[:, 32:]


_embed = pl.pallas_call(
    _embed_body, grid=(_GN,),
    in_specs=[pl.BlockSpec((1, 1, _BN), lambda g: (g, 0, 0)),
              _w_spec((100, _H)), _w_spec((100, _H))],
    out_specs=[pl.BlockSpec((_BN, _H), lambda g: (g, 0)),
               pl.BlockSpec((2, _BN, 32), lambda g: (0, g, 0))],
    out_shape=[jax.ShapeDtypeStruct((_N, _H), jnp.float32),
               jax.ShapeDtypeStruct((2, _N, 32), jnp.float32)])


def _comb_body(x0_ref, agg_ref, cw_ref, cb_ref, l1_ref, x_ref, x1_ref):
    cat = jnp.concatenate([x0_ref[...], agg_ref[0], agg_ref[1]], axis=1)
    xb = jnp.dot(cat, cw_ref[...], preferred_element_type=jnp.float32) + cb_ref[0, :][None, :]
    x_ref[...] = xb
    x1 = jnp.dot(xb, l1_ref[...], preferred_element_type=jnp.float32)
    x1_ref[0] = x1[:, :32]
    x1_ref[1] = x1[:, 32:]


_comb = pl.pallas_call(
    _comb_body, grid=(_GN,),
    in_specs=[pl.BlockSpec((_BN, _H), lambda g: (g, 0)),
              pl.BlockSpec((2, _BN, 32), lambda g: (0, g, 0)),
              _w_spec((2 * _H, _H)), _w_spec((1, _H)), _w_spec((_H, _H))],
    out_specs=[pl.BlockSpec((_BN, _H), lambda g: (g, 0)),
               pl.BlockSpec((2, _BN, 32), lambda g: (0, g, 0))],
    out_shape=[jax.ShapeDtypeStruct((_N, _H), jnp.float32),
               jax.ShapeDtypeStruct((2, _N, 32), jnp.float32)])


def _layer_body(x_ref, s_ref, hist_ref, l2_ref, l2b_ref, lw_ref, lwb_ref, *rest,
                has_next):
    if has_next:
        l1n_ref, x_out, x1_out = rest
    else:
        (x_out,) = rest
    cnt = jnp.clip(hist_ref[0, :, 0] + hist_ref[1, :, 0], 1.0, None)
    sm = jnp.concatenate([s_ref[0], s_ref[1]], axis=1) / cnt[:, None]
    v = jnp.dot(sm, l2_ref[...], preferred_element_type=jnp.float32) + l2b_ref[0, :][None, :]
    v = jax.nn.silu(v)
    v = jnp.dot(v, lw_ref[...], preferred_element_type=jnp.float32) + lwb_ref[0, :][None, :]
    xn = x_ref[...] + v
    x_out[...] = xn
    if has_next:
        x1 = jnp.dot(xn, l1n_ref[...], preferred_element_type=jnp.float32)
        x1_out[0] = x1[:, :32]
        x1_out[1] = x1[:, 32:]


def _make_layer(has_next):
    in_specs = [pl.BlockSpec((_BN, _H), lambda g: (g, 0)),
                pl.BlockSpec((2, _BN, 32), lambda g: (0, g, 0)),
                pl.BlockSpec((2, _BN, 8), lambda g: (0, g, 0)),
                _w_spec((_H, _H)), _w_spec((1, _H)),
                _w_spec((_H, _H)), _w_spec((1, _H))]
    out_specs = [pl.BlockSpec((_BN, _H), lambda g: (g, 0))]
    out_shape = [jax.ShapeDtypeStruct((_N, _H), jnp.float32)]
    if has_next:
        in_specs.append(_w_spec((_H, _H)))
        out_specs.append(pl.BlockSpec((2, _BN, 32), lambda g: (0, g, 0)))
        out_shape.append(jax.ShapeDtypeStruct((2, _N, 32), jnp.float32))
    return pl.pallas_call(
        functools.partial(_layer_body, has_next=has_next), grid=(_GN,),
        in_specs=in_specs, out_specs=out_specs, out_shape=out_shape)


_layer_next = _make_layer(True)
_layer_last = _make_layer(False)


_SC_PARAMS = pltpu.CompilerParams(needs_layout_passes=False,
                                  use_tc_tiling_on_sc=False)


def _make_sc_scatter():
    mesh = plsc.VectorSubcoreMesh(core_axis_name="c", subcore_axis_name="s",
                                  num_cores=_NC)
    out_type = jax.ShapeDtypeStruct((_NC * _NP, 32), jnp.float32)
    scratch = [
        pltpu.VMEM((8, 128), jnp.int32),
        pltpu.VMEM((8, 128), jnp.int32),
        pltpu.VMEM((2, 128, 32), jnp.float32),
        pltpu.VMEM((2, 128, 32), jnp.float32),
        pltpu.VMEM_SHARED((_NP, 32), jnp.float32),
        pltpu.SemaphoreType.DMA, pltpu.SemaphoreType.DMA,
        pltpu.SemaphoreType.DMA, pltpu.SemaphoreType.DMA,
        pltpu.SemaphoreType.DMA, pltpu.SemaphoreType.DMA,
    ]

    def body(src_hbm, dst_hbm, wf_hbm, table_hbm, zeros_hbm, out_hbm,
             srcv, dstv, gath, wfv, acc, sg0, sg1, sw0, sw1, ss0, ss1):
        c = lax.axis_index("c")
        t = lax.axis_index("s")
        sgs, sws, sss = [sg0, sg1], [sw0, sw1], [ss0, ss1]

        pltpu.sync_copy(zeros_hbm, acc.at[pl.ds(t * _RPT, _RPT)])
        plsc.subcore_barrier()

        idx_row0 = t * (_EPT // 128)

        def grp(i, carry):
            rb = idx_row0 + i * 8
            pltpu.sync_copy(src_hbm.at[pl.ds(c * (_EPAD // 128) + rb, 8)], srcv)
            pltpu.sync_copy(dst_hbm.at[pl.ds(rb, 8)], dstv)
            gbase = t * _EPT + i * 1024

            def issue(sb):
                s = sb % 2
                rowbase = (t * 50 + i) * 256 + (sb % 2) * 128
                hw = pltpu.async_copy(
                    wf_hbm.at[c, pl.ds(rowbase, 128),
                              pl.ds((sb // 2) * 32, 32)],
                    wfv.at[s], sws[s])
                hg = pltpu.async_copy(table_hbm.at[srcv.at[sb]],
                                      gath.at[s], sgs[s])
                return hw, hg

            hws, hgs = [None] * 8, [None] * 8
            hss = [None] * 8
            hws[0], hgs[0] = issue(0)
            for sb in range(8):
                s = sb % 2
                if sb < 7:
                    if sb >= 1:
                        hss[sb - 1].wait()
                    hws[sb + 1], hgs[sb + 1] = issue(sb + 1)
                hws[sb].wait()
                hgs[sb].wait()
                gslot, wslot = gath.at[s], wfv.at[s]

                @plsc.parallel_loop(0, 128, unroll=8)
                def _(r):
                    gslot[r, pl.ds(0, 16)] = (gslot[r, pl.ds(0, 16)]
                                              * wslot[r, pl.ds(0, 16)])
                    gslot[r, pl.ds(16, 16)] = (gslot[r, pl.ds(16, 16)]
                                               * wslot[r, pl.ds(16, 16)])

                hss[sb] = pltpu.async_copy(gath.at[s], acc.at[dstv.at[sb]],
                                           sss[s], add=True)
            hss[6].wait()
            hss[7].wait()
            return carry

        lax.fori_loop(0, _EPT // 1024, grp, 0)
        plsc.subcore_barrier()
        pltpu.sync_copy(acc.at[pl.ds(t * _RPT, _RPT)],
                        out_hbm.at[pl.ds(c * _NP + t * _RPT, _RPT)])

    return pl.kernel(body, out_type=out_type, mesh=mesh, scratch_types=scratch,
                     compiler_params=_SC_PARAMS)


def _make_sc_degree():
    mesh = plsc.VectorSubcoreMesh(core_axis_name="c", subcore_axis_name="s",
                                  num_cores=_NC)
    out_type = jax.ShapeDtypeStruct((_NC * _NP, 8), jnp.float32)
    scratch = [
        pltpu.VMEM((8, 128), jnp.int32),
        pltpu.VMEM((128, 8), jnp.float32),
        pltpu.VMEM_SHARED((_NP, 8), jnp.float32),
        pltpu.SemaphoreType.DMA, pltpu.SemaphoreType.DMA,
        pltpu.SemaphoreType.DMA, pltpu.SemaphoreType.DMA,
    ]
    rows_per_worker = (_EPAD // 128) // (_NC * _NS)

    def body(dst_hbm, zeros8_hbm, ones_hbm, out_hbm, dstv, ones, hist,
             s0, s1, s2, s3):
        c = lax.axis_index("c")
        t = lax.axis_index("s")
        sems = [s0, s1, s2, s3]
        pltpu.sync_copy(zeros8_hbm, hist.at[pl.ds(t * _RPT, _RPT)])
        pltpu.sync_copy(ones_hbm, ones)
        plsc.subcore_barrier()

        row0 = (c * _NS + t) * rows_per_worker

        def grp(i, carry):
            pltpu.sync_copy(dst_hbm.at[pl.ds(row0 + i * 8, 8)], dstv)
            hs = [pltpu.async_copy(ones, hist.at[dstv.at[j]], sems[j % 4],
                                   add=True)
                  for j in range(8)]
            for h in hs:
                h.wait()
            return carry

        lax.fori_loop(0, rows_per_worker // 8, grp, 0)
        plsc.subcore_barrier()
        pltpu.sync_copy(hist.at[pl.ds(t * _RPT, _RPT)],
                        out_hbm.at[pl.ds(c * _NP + t * _RPT, _RPT)])

    return pl.kernel(body, out_type=out_type, mesh=mesh, scratch_types=scratch,
                     compiler_params=_SC_PARAMS)


_sc_scatter = _make_sc_scatter()
_sc_degree = _make_sc_degree()


def kernel(z, edge_index, edge_weight, emb, emb2, means, betas, dp_W, dp_b,
           comb_W, comb_b, mlp1_W, mlp1_b, mlp2_W, mlp2_b,
           lin1_W, lin2_W, lin2_b, lin_W, lin_b):
    f32 = jnp.float32
    src = edge_index[0].astype(jnp.int32)
    dst = edge_index[1].astype(jnp.int32)
    pad_idx = (jnp.arange(_PAD, dtype=jnp.int32) * 97) % _N
    src_p = jnp.concatenate([src, pad_idx])
    dst_p = jnp.concatenate([dst, jnp.full((_PAD,), _N, jnp.int32)])
    src2 = jnp.concatenate([src_p, src_p + _N]).reshape(2 * _EPAD // 128, 128)
    dst2 = dst_p.reshape(_EPAD // 128, 128)
    d_pad = jnp.concatenate(
        [edge_weight.astype(f32), jnp.full((_PAD,), 7.0, f32)]).reshape(_GE, 1, _BE)
    zeros2d = jnp.zeros((_RPT, 32), f32)
    zeros8 = jnp.zeros((_RPT, 8), f32)
    ones8 = jnp.ones((128, 8), f32)
    means2 = means.reshape(1, _R)
    betas2 = betas.reshape(1, _R)

    x0, xn = _embed(z.astype(jnp.int32).reshape(_GN, 1, _BN), emb, emb2)

    wn = _filter1(d_pad, means2, betas2, dp_W, dp_b.reshape(1, _H))
    agg_flat = _sc_scatter(src2, dst2, wn, xn.reshape(2 * _N, 32), zeros2d)
    agg = agg_flat.reshape(2, _NP, 32)[:, :_N, :]
    hist = _sc_degree(dst2, zeros8, ones8)
    hist3 = hist.reshape(2, _NP, 8)[:, :_N, :]

    x, x1 = _comb(x0, agg, comb_W, comb_b.reshape(1, _H), lin1_W[0])

    for l in range(4):
        wf = _filter2(d_pad, means2, betas2, mlp1_W[l], mlp1_b[l].reshape(1, _H),
                      mlp2_W[l], mlp2_b[l].reshape(1, _H))
        s_flat = _sc_scatter(src2, dst2, wf, x1.reshape(2 * _N, 32), zeros2d)
        s = s_flat.reshape(2, _NP, 32)[:, :_N, :]
        common = (x, s, hist3, lin2_W[l], lin2_b[l].reshape(1, _H),
                  lin_W[l], lin_b[l].reshape(1, _H))
        if l < 3:
            x, x1 = _layer_next(*common, lin1_W[l + 1])
        else:
            x, = _layer_last(*common)
    return x
